# Initial kernel scaffold; baseline (speedup 1.0000x reference)
import jax
import jax.numpy as jnp
from jax.experimental import pallas as pl


def kernel(edge_index, edge_weight):
    raise NotImplementedError("write your pallas kernel here")


# speedup vs baseline: 11.6258x; 11.6258x over previous
import functools

import jax
import jax.numpy as jnp
from jax import lax
from jax.experimental import pallas as pl
from jax.experimental.pallas import tpu as pltpu
from jax.experimental.pallas import tpu_sc as plsc

N_NODES = 100000
N_EDGES = 6400000

NC = 2
NS = 16
L = 16
NW = NC * NS
EPW = N_EDGES // NW
CH = 4000
NCH = EPW // CH
NB_T = 25
NPT = N_NODES // NB_T

_MESH = dict(core_axis_name="c", subcore_axis_name="s", num_cores=NC,
             num_subcores=NS)


def _wid():
    return lax.axis_index("s") * NC + lax.axis_index("c")


def _iota16():
    return lax.iota(jnp.int32, L)


def _rsqrt16(x):
    xi = plsc.bitcast(x, jnp.int32)
    yi = jnp.full((L,), 0x5F3759DF, jnp.int32) - lax.shift_right_logical(
        xi, jnp.full((L,), 1, jnp.int32))
    y = plsc.bitcast(yi, jnp.float32)
    half = jnp.full((L,), 0.5, jnp.float32)
    three_half = jnp.full((L,), 1.5, jnp.float32)
    for _ in range(3):
        y = y * (three_half - half * x * y * y)
    return y


def _deg_body(ei_hbm, ew_hbm, part_hbm, ebuf, wbuf, deg):
    wid = _wid()
    zeros16 = jnp.zeros((L,), jnp.float32)

    def zinit(i, _):
        deg[pl.ds(i * L, L)] = zeros16
        return 0
    lax.fori_loop(0, N_NODES // L, zinit, 0)

    iota2 = _iota16() * 2

    def do_chunk(c, _):
        base = wid * EPW + c * CH
        pltpu.sync_copy(ei_hbm.at[pl.ds(base * 2, CH * 2)], ebuf)
        pltpu.sync_copy(ew_hbm.at[pl.ds(base, CH)], wbuf)

        def body(j, _):
            ridx = j * (2 * L) + iota2
            rows = plsc.load_gather(ebuf, [ridx])
            w = wbuf[pl.ds(j * L, L)]
            plsc.addupdate_scatter(deg, [rows], w)
            return 0
        lax.fori_loop(0, CH // L, body, 0)
        return 0
    lax.fori_loop(0, NCH, do_chunk, 0)

    pltpu.sync_copy(deg, part_hbm.at[pl.ds(wid * N_NODES, N_NODES)])


@functools.partial(jax.jit, donate_argnums=())
def _deg_kernel(edge_index, edge_weight):
    return pl.kernel(
        _deg_body,
        out_type=jax.ShapeDtypeStruct((NW * N_NODES,), jnp.float32),
        mesh=plsc.VectorSubcoreMesh(**_MESH),
        compiler_params=pltpu.CompilerParams(needs_layout_passes=False),
        scratch_types=[
            pltpu.VMEM((CH * 2,), jnp.int32),
            pltpu.VMEM((CH,), jnp.float32),
            pltpu.VMEM((N_NODES,), jnp.float32),
        ],
    )(edge_index.reshape(N_EDGES * 2), edge_weight)


def _reduce_body(part_hbm, dis_hbm, acc, buf, disb):
    wid = _wid()

    @pl.when(wid < NB_T)
    def _():
        base = wid * NPT
        ones16 = jnp.full((L,), 1.0, jnp.float32)

        def init(i, _):
            acc[pl.ds(i * L, L)] = ones16
            return 0
        lax.fori_loop(0, NPT // L, init, 0)

        for k in range(NW):
            pltpu.sync_copy(part_hbm.at[pl.ds(k * N_NODES + base, NPT)], buf)

            def add(i, _):
                s = pl.ds(i * L, L)
                acc[s] = acc[s] + buf[s]
                return 0
            lax.fori_loop(0, NPT // L, add, 0)

        def finish(i, _):
            s = pl.ds(i * L, L)
            disb[s] = _rsqrt16(acc[s])
            return 0
        lax.fori_loop(0, NPT // L, finish, 0)

        pltpu.sync_copy(disb, dis_hbm.at[pl.ds(base, NPT)])


@jax.jit
def _reduce_kernel(part):
    return pl.kernel(
        _reduce_body,
        out_type=jax.ShapeDtypeStruct((N_NODES,), jnp.float32),
        mesh=plsc.VectorSubcoreMesh(**_MESH),
        compiler_params=pltpu.CompilerParams(needs_layout_passes=False),
        scratch_types=[
            pltpu.VMEM((NPT,), jnp.float32),
            pltpu.VMEM((NPT,), jnp.float32),
            pltpu.VMEM((NPT,), jnp.float32),
        ],
    )(part)


def _norm_body(ei_hbm, ew_hbm, dis_hbm, out_hbm, disb, ebuf, wbuf, obuf):
    wid = _wid()
    pltpu.sync_copy(dis_hbm, disb)

    iota2 = _iota16() * 2
    ones16i = jnp.ones((L,), jnp.int32)

    @pl.when(wid < NB_T)
    def _():
        def sbody(i, _):
            v = disb[pl.ds(wid * NPT + i * L, L)]
            obuf[pl.ds(i * L, L)] = v * v
            return 0
        lax.fori_loop(0, NPT // L, sbody, 0)
        pltpu.sync_copy(obuf, out_hbm.at[pl.ds(N_EDGES + wid * NPT, NPT)])

    def do_chunk(c, _):
        base = wid * EPW + c * CH
        pltpu.sync_copy(ei_hbm.at[pl.ds(base * 2, CH * 2)], ebuf)
        pltpu.sync_copy(ew_hbm.at[pl.ds(base, CH)], wbuf)

        def body(j, _):
            ridx = j * (2 * L) + iota2
            rows = plsc.load_gather(ebuf, [ridx])
            cols = plsc.load_gather(ebuf, [ridx + ones16i])
            dr = plsc.load_gather(disb, [rows])
            dc = plsc.load_gather(disb, [cols])
            w = wbuf[pl.ds(j * L, L)]
            obuf[pl.ds(j * L, L)] = dr * w * dc
            return 0
        lax.fori_loop(0, CH // L, body, 0)

        pltpu.sync_copy(obuf, out_hbm.at[pl.ds(base, CH)])
        return 0
    lax.fori_loop(0, NCH, do_chunk, 0)


@jax.jit
def _norm_kernel(edge_index, edge_weight, dis):
    return pl.kernel(
        _norm_body,
        out_type=jax.ShapeDtypeStruct((N_EDGES + N_NODES,), jnp.float32),
        mesh=plsc.VectorSubcoreMesh(**_MESH),
        compiler_params=pltpu.CompilerParams(needs_layout_passes=False),
        scratch_types=[
            pltpu.VMEM((N_NODES,), jnp.float32),
            pltpu.VMEM((CH * 2,), jnp.int32),
            pltpu.VMEM((CH,), jnp.float32),
            pltpu.VMEM((CH,), jnp.float32),
        ],
    )(edge_index.reshape(N_EDGES * 2), edge_weight, dis)


def kernel(edge_index, edge_weight):
    diag = jnp.arange(N_NODES, dtype=edge_index.dtype)
    ei = jnp.concatenate(
        [edge_index, jnp.stack([diag, diag], axis=1)], axis=0)
    part = _deg_kernel(edge_index, edge_weight)
    dis = _reduce_kernel(part)
    normed = _norm_kernel(edge_index, edge_weight, dis)
    return (ei, normed)
